# Initial kernel scaffold; baseline (speedup 1.0000x reference)
import jax
import jax.numpy as jnp
from jax.experimental import pallas as pl


def kernel(input_ids, token_type_ids, seg_table, pos_table, row_table, col_table):
    raise NotImplementedError("write your pallas kernel here")


# speedup vs baseline: 1.4007x; 1.4007x over previous
import functools

import jax
import jax.numpy as jnp
from jax import lax
from jax.experimental import pallas as pl
from jax.experimental.pallas import tpu as pltpu
from jax.experimental.pallas import tpu_sc as plsc

LANES = 16


@functools.lru_cache(maxsize=None)
def _make_sc_kernel(n_rows, seq, hidden, n_types):
    info = plsc.get_sparse_core_info()
    nc, ns = info.num_cores, info.num_subcores
    nw = nc * ns
    assert n_rows % nw == 0
    rows_per_w = n_rows // nw
    assert seq % rows_per_w == 0
    CH = LANES
    n_chunks = rows_per_w // CH
    nh = hidden // LANES
    assert hidden % LANES == 0

    mesh = plsc.VectorSubcoreMesh(core_axis_name="c", subcore_axis_name="s")

    def body(tt_hbm, seg_hbm, rowt_hbm, colt_hbm, pos_hbm, out_hbm,
             comb_v, aux0, aux1, tt_v):
        cid = lax.axis_index("c")
        sid = lax.axis_index("s")
        wid = sid * nc + cid
        row_base = wid * rows_per_w
        s_base = lax.rem(row_base, seq)

        pltpu.sync_copy(seg_hbm, comb_v)
        pltpu.sync_copy(rowt_hbm, aux0)
        pltpu.sync_copy(colt_hbm, aux1)
        pltpu.sync_copy(tt_hbm.at[pl.ds(row_base, rows_per_w)], tt_v)

        def combine_row(i, carry):
            for j in range(nh):
                jds = pl.ds(j * LANES, LANES)
                comb_v[i, jds] = comb_v[i, jds] + aux0[i, jds] + aux1[i, jds]
            return carry

        lax.fori_loop(0, n_types, combine_row, 0)

        def chunk(c, carry):
            pltpu.sync_copy(pos_hbm.at[pl.ds(s_base + c * CH, CH)],
                            aux0.at[pl.ds(0, CH)])
            ttvec = tt_v[pl.ds(c * CH, CH)]
            for r in range(CH):
                t = ttvec[r]
                for j in range(nh):
                    jds = pl.ds(j * LANES, LANES)
                    aux0[r, jds] = aux0[r, jds] + comb_v[t, jds]
            pltpu.sync_copy(aux0.at[pl.ds(0, CH)],
                            out_hbm.at[pl.ds(row_base + c * CH, CH)])
            return carry

        lax.fori_loop(0, n_chunks, chunk, 0)

    return pl.kernel(
        body,
        out_type=jax.ShapeDtypeStruct((n_rows, hidden), jnp.float32),
        mesh=mesh,
        scratch_types=[
            pltpu.VMEM((n_types, hidden), jnp.float32),
            pltpu.VMEM((n_types, hidden), jnp.float32),
            pltpu.VMEM((n_types, hidden), jnp.float32),
            pltpu.VMEM((rows_per_w,), jnp.int32),
        ],
    )


def kernel(input_ids, token_type_ids, seg_table, pos_table, row_table,
           col_table):
    batch, seq = token_type_ids.shape
    n_types, hidden = seg_table.shape
    tt = token_type_ids.astype(jnp.int32).reshape(-1)
    sc = _make_sc_kernel(batch * seq, seq, hidden, n_types)
    out = sc(tt, seg_table, row_table, col_table, pos_table)
    return out.reshape(batch, seq, hidden)


# speedup vs baseline: 2.9395x; 2.0986x over previous
import functools

import jax
import jax.numpy as jnp
from jax import lax
from jax.experimental import pallas as pl
from jax.experimental.pallas import tpu as pltpu
from jax.experimental.pallas import tpu_sc as plsc

LANES = 16
NSLOTS = 4


@functools.lru_cache(maxsize=None)
def _make_sc_kernel(n_rows, seq, hidden, n_types):
    info = plsc.get_sparse_core_info()
    nc, ns = info.num_cores, info.num_subcores
    nw = nc * ns
    assert n_rows % nw == 0
    rows_per_w = n_rows // nw
    assert seq % rows_per_w == 0
    CH = LANES
    n_chunks = rows_per_w // CH
    nh = hidden // LANES
    assert hidden % LANES == 0
    assert n_chunks % NSLOTS == 0 and n_chunks >= 2 * NSLOTS
    assert n_types == 2 * CH

    mesh = plsc.VectorSubcoreMesh(core_axis_name="c", subcore_axis_name="s")

    def body(tt_hbm, seg_hbm, rowt_hbm, colt_hbm, pos_hbm, out_hbm,
             comb_v, b0, b1, b2, b3, tt_v,
             is0, is1, is2, is3, os0, os1, os2, os3):
        bufs = (b0, b1, b2, b3)
        in_sems = (is0, is1, is2, is3)
        out_sems = (os0, os1, os2, os3)

        cid = lax.axis_index("c")
        sid = lax.axis_index("s")
        wid = sid * nc + cid
        row_base = wid * rows_per_w
        s_base = lax.rem(row_base, seq)

        pltpu.sync_copy(seg_hbm, comb_v)
        pltpu.sync_copy(rowt_hbm.at[pl.ds(0, CH)], b0)
        pltpu.sync_copy(rowt_hbm.at[pl.ds(CH, CH)], b1)
        pltpu.sync_copy(colt_hbm.at[pl.ds(0, CH)], b2)
        pltpu.sync_copy(colt_hbm.at[pl.ds(CH, CH)], b3)
        pltpu.sync_copy(tt_hbm.at[pl.ds(row_base, rows_per_w)], tt_v)

        def combine_row(i, carry):
            for j in range(nh):
                jds = pl.ds(j * LANES, LANES)
                comb_v[i, jds] = comb_v[i, jds] + b0[i, jds] + b2[i, jds]
                comb_v[i + CH, jds] = (comb_v[i + CH, jds] + b1[i, jds]
                                       + b3[i, jds])
            return carry

        lax.fori_loop(0, CH, combine_row, 0)

        def in_copy(c, k):
            return pltpu.make_async_copy(
                pos_hbm.at[pl.ds(s_base + c * CH, CH)], bufs[k], in_sems[k])

        def out_copy(c, k):
            return pltpu.make_async_copy(
                bufs[k], out_hbm.at[pl.ds(row_base + c * CH, CH)],
                out_sems[k])

        in_copy(0, 0).start()
        in_copy(1, 1).start()

        def step(g, carry):
            for k in range(NSLOTS):
                c = g * NSLOTS + k
                in_copy(c, k).wait()
                ttvec = tt_v[pl.ds(c * CH, CH)]
                ts = [ttvec[r] for r in range(CH)]
                buf = bufs[k]

                def jbody(j, carry2):
                    jds = pl.ds(j * LANES, LANES)
                    for r in range(CH):
                        plsc.addupdate(buf.at[r, jds], comb_v[ts[r], jds])
                    return carry2

                lax.fori_loop(0, nh, jbody, 0)
                out_copy(c, k).start()

                kp = (k + 2) % NSLOTS
                if k < 2:
                    @pl.when(g >= 1)
                    def _wait():
                        out_copy(c + 2 - NSLOTS, kp).wait()
                    in_copy(c + 2, kp).start()
                else:
                    @pl.when(g < (n_chunks // NSLOTS) - 1)
                    def _pre():
                        out_copy(c + 2 - NSLOTS, kp).wait()
                        in_copy(c + 2, kp).start()
            return carry

        lax.fori_loop(0, n_chunks // NSLOTS, step, 0)

        for k in range(NSLOTS):
            out_copy(n_chunks - NSLOTS + k, k).wait()

    return pl.kernel(
        body,
        out_type=jax.ShapeDtypeStruct((n_rows, hidden), jnp.float32),
        mesh=mesh,
        scratch_types=(
            [pltpu.VMEM((n_types, hidden), jnp.float32)]
            + [pltpu.VMEM((CH, hidden), jnp.float32)] * NSLOTS
            + [pltpu.VMEM((rows_per_w,), jnp.int32)]
            + [pltpu.SemaphoreType.DMA] * (2 * NSLOTS)
        ),
    )


def kernel(input_ids, token_type_ids, seg_table, pos_table, row_table,
           col_table):
    batch, seq = token_type_ids.shape
    n_types, hidden = seg_table.shape
    tt = token_type_ids.astype(jnp.int32).reshape(-1)
    sc = _make_sc_kernel(batch * seq, seq, hidden, n_types)
    out = sc(tt, seg_table, row_table, col_table, pos_table)
    return out.reshape(batch, seq, hidden)
